# Initial kernel scaffold; baseline (speedup 1.0000x reference)
import jax
import jax.numpy as jnp
from jax.experimental import pallas as pl


def kernel(x, emb, W2, b2, W3, b3, W4):
    raise NotImplementedError("write your pallas kernel here")


# speedup vs baseline: 4.7650x; 4.7650x over previous
import functools

import jax
import jax.numpy as jnp
from jax import lax
from jax.experimental import pallas as pl
from jax.experimental.pallas import tpu as pltpu
from jax.experimental.pallas import tpu_sc as plsc


def _sc_gather_sum(x_flat, emb, *, interpret=False):
    total_idx = x_flat.shape[0]
    B = total_idx // 32
    D = emb.shape[1]
    NC, NS = 2, 16
    NW = NC * NS
    BPW = B // NW
    CH = 4
    ROWS = CH * 32
    NCHUNK = BPW // CH
    ND = D // 16

    mesh = plsc.VectorSubcoreMesh(core_axis_name="c", subcore_axis_name="s")

    @functools.partial(
        pl.kernel,
        out_type=jax.ShapeDtypeStruct((B, D), jnp.float32),
        mesh=mesh,
        scratch_types=[
            pltpu.VMEM((BPW * 32,), jnp.int32),
            pltpu.VMEM((2, ROWS, D), jnp.float32),
            pltpu.VMEM((BPW, D), jnp.float32),
            pltpu.SemaphoreType.DMA,
            pltpu.SemaphoreType.DMA,
        ],
        interpret=interpret,
    )
    def gather_kernel(x_hbm, emb_hbm, out_hbm, x_v, rows_v, out_v, sem0, sem1):
        wid = lax.axis_index("s") * NC + lax.axis_index("c")
        base = wid * BPW
        sems = (sem0, sem1)

        pltpu.sync_copy(x_hbm.at[pl.ds(base * 32, BPW * 32)], x_v)

        def issue(c, b):
            pltpu.async_copy(
                emb_hbm.at[x_v.at[pl.ds(c * ROWS, ROWS)]],
                rows_v.at[b],
                sems[b],
            )

        def wait(b):
            pltpu.make_async_copy(
                emb_hbm.at[x_v.at[pl.ds(0, ROWS)]],
                rows_v.at[b],
                sems[b],
            ).wait()

        issue(0, 0)
        issue(1, 1)

        def outer(g, _):
            for b in range(2):
                c = g * 2 + b
                wait(b)
                for e in range(CH):
                    r0 = e * 32
                    for d in range(ND):
                        sl = pl.ds(d * 16, 16)
                        acc = rows_v[b, r0, sl]
                        for j in range(1, 29):
                            acc = acc + rows_v[b, r0 + j, sl]
                        out_v[c * CH + e, sl] = acc
                nxt = c + 2

                @pl.when(nxt < NCHUNK)
                def _():
                    issue(nxt, b)
            return _

        lax.fori_loop(0, NCHUNK // 2, outer, None)
        pltpu.sync_copy(out_v, out_hbm.at[pl.ds(base, BPW)])

    return gather_kernel(x_flat, emb)


def _mlp_body(pooled_ref, w2_ref, b2_ref, w3_ref, b3_ref, w4_ref, out_ref):
    h = jnp.maximum(pooled_ref[...], 0.0)
    a = lax.dot_general(h, w2_ref[...], (((1,), (1,)), ((), ())),
                        preferred_element_type=jnp.float32)
    a = a + b2_ref[...][None, :]
    h = jnp.concatenate((a, -a), axis=-1)
    h = jnp.maximum(h, 0.0)
    a = lax.dot_general(h, w3_ref[...], (((1,), (1,)), ((), ())),
                        preferred_element_type=jnp.float32)
    a = a + b3_ref[...][None, :]
    h = jnp.concatenate((a, -a), axis=-1)
    h = jnp.maximum(h, 0.0)
    out_ref[...] = lax.dot_general(h, w4_ref[...], (((1,), (1,)), ((), ())),
                                   preferred_element_type=jnp.float32)


def _tc_mlp(pooled, W2, b2, W3, b3, W4, *, interpret=False):
    B, D = pooled.shape
    BB = 2048
    grid = (B // BB,)
    return pl.pallas_call(
        _mlp_body,
        grid=grid,
        in_specs=[
            pl.BlockSpec((BB, D), lambda i: (i, 0)),
            pl.BlockSpec(W2.shape, lambda i: (0, 0)),
            pl.BlockSpec(b2.shape, lambda i: (0,)),
            pl.BlockSpec(W3.shape, lambda i: (0, 0)),
            pl.BlockSpec(b3.shape, lambda i: (0,)),
            pl.BlockSpec(W4.shape, lambda i: (0, 0)),
        ],
        out_specs=pl.BlockSpec((BB, 1), lambda i: (i, 0)),
        out_shape=jax.ShapeDtypeStruct((B, 1), jnp.float32),
        interpret=interpret,
    )(pooled, W2, b2, W3, b3, W4)


def kernel(x, emb, W2, b2, W3, b3, W4):
    x_flat = x.astype(jnp.int32).reshape(-1)
    pooled = _sc_gather_sum(x_flat, emb)
    return _tc_mlp(pooled, W2, b2, W3, b3, W4)
